# baseline (speedup 1.0000x reference)
import jax
import jax.numpy as jnp
from jax import lax
from jax.experimental import pallas as pl
from jax.experimental.pallas import tpu as pltpu
from jax.experimental.pallas import tpu_sc as plsc

E = 320000
D = 128
PW = D // 2
HPW = PW // 2
N = 10000
N2 = 2 * N
NC = 2
NS = 16
EPT = E // NS
C = 80
NCHUNK = EPT // C
G = C // 16
NBUF = 2
NSTAGE = 10
RPS = N2 // NSTAGE


def _sc_body(h0_hbm, h1_hbm, src_hbm, dst_hbm, p0_hbm, p1_hbm,
             sidx0, sidx1, didx0, didx1, srow0, srow1, drow0, drow1,
             part_v, zh_sh, semr0, semr1, semi0, semi1):
    sidx = (sidx0, sidx1)
    didx = (didx0, didx1)
    srow = (srow0, srow1)
    drow = (drow0, drow1)
    semr = (semr0, semr1)
    semi = (semi0, semi1)

    cid = lax.axis_index("c")
    sid = lax.axis_index("s")
    base = sid * EPT

    @pl.when(sid < NSTAGE)
    def _():
        row0 = sid * RPS

        @pl.when(cid == 0)
        def _():
            pltpu.sync_copy(h0_hbm.at[pl.ds(row0, RPS)],
                            zh_sh.at[pl.ds(row0, RPS)])

        @pl.when(cid == 1)
        def _():
            pltpu.sync_copy(h1_hbm.at[pl.ds(row0, RPS)],
                            zh_sh.at[pl.ds(row0, RPS)])

    plsc.subcore_barrier()

    def fire_idx(ci, b):
        cbase = base + ci * C
        pltpu.async_copy(src_hbm.at[pl.ds(cbase, C)], sidx[b], semi[b])
        pltpu.async_copy(dst_hbm.at[pl.ds(cbase, C)], didx[b], semi[b])

    def drain_idx(b):
        pltpu.make_async_copy(src_hbm.at[pl.ds(0, C)], sidx[b], semi[b]).wait()
        pltpu.make_async_copy(dst_hbm.at[pl.ds(0, C)], didx[b], semi[b]).wait()

    def fire_rows(b):
        pltpu.async_copy(zh_sh.at[sidx[b]], srow[b], semr[b])
        pltpu.async_copy(zh_sh.at[didx[b]], drow[b], semr[b])

    def drain_rows(b):
        pltpu.make_async_copy(zh_sh.at[sidx[b]], srow[b], semr[b]).wait()
        pltpu.make_async_copy(zh_sh.at[didx[b]], drow[b], semr[b]).wait()

    def compute(ci, b):
        for g in range(G):
            lanes = lax.iota(jnp.int32, 16) + (g * 16)
            acca0 = jnp.zeros((16,), jnp.float32)
            accb0 = jnp.zeros((16,), jnp.float32)
            dv0 = jnp.zeros((16,), jnp.int32)

            @pl.loop(0, HPW, init_carry=(acca0, accb0, dv0), unroll=8)
            def dot_loop(d, carry):
                acca, accb, dv = carry
                si = plsc.load_gather(srow[b], [lanes, dv])
                ti = plsc.load_gather(drow[b], [lanes, dv])
                sbf = plsc.bitcast(si, jnp.bfloat16)
                tbf = plsc.bitcast(ti, jnp.bfloat16)
                q0, q1 = plsc.unpack(sbf * tbf,
                                     format=plsc.PackFormat.INTERLEAVED)
                return acca + q0, accb + q1, dv + 1

            acca, accb, _ = dot_loop
            part_v[pl.ds(ci * C + g * 16, 16)] = acca + accb

    fire_idx(0, 0)
    drain_idx(0)
    fire_rows(0)
    fire_idx(1, 1)

    @pl.loop(0, NCHUNK, step=NBUF)
    def chunk_loop(ci0):
        for b in range(NBUF):
            ci = ci0 + b
            bn = 1 - b

            @pl.when(ci + 1 < NCHUNK)
            def _():
                drain_idx(bn)
                fire_rows(bn)

            drain_rows(b)

            @pl.when(ci + NBUF < NCHUNK)
            def _():
                fire_idx(ci + NBUF, b)

            compute(ci, b)

    @pl.when(cid == 0)
    def _():
        pltpu.sync_copy(part_v, p0_hbm.at[pl.ds(base, EPT)])

    @pl.when(cid == 1)
    def _():
        pltpu.sync_copy(part_v, p1_hbm.at[pl.ds(base, EPT)])


def _sigmoid_body(p0_ref, p1_ref, out_ref):
    out_ref[...] = jax.nn.sigmoid(p0_ref[...] + p1_ref[...])


@jax.jit
def _edge_decoder(h0_pk, h1_pk, src_idx, dst_idx):
    mesh = plsc.VectorSubcoreMesh(
        core_axis_name="c", subcore_axis_name="s",
        num_cores=NC, num_subcores=NS)
    p0, p1 = pl.kernel(
        _sc_body,
        out_type=(jax.ShapeDtypeStruct((E,), jnp.float32),
                  jax.ShapeDtypeStruct((E,), jnp.float32)),
        mesh=mesh,
        compiler_params=pltpu.CompilerParams(
            needs_layout_passes=False, use_tc_tiling_on_sc=False),
        scratch_types=[
            pltpu.VMEM((C,), jnp.int32),
            pltpu.VMEM((C,), jnp.int32),
            pltpu.VMEM((C,), jnp.int32),
            pltpu.VMEM((C,), jnp.int32),
            pltpu.VMEM((C, HPW), jnp.int32),
            pltpu.VMEM((C, HPW), jnp.int32),
            pltpu.VMEM((C, HPW), jnp.int32),
            pltpu.VMEM((C, HPW), jnp.int32),
            pltpu.VMEM((EPT,), jnp.float32),
            pltpu.VMEM_SHARED((N2, HPW), jnp.int32),
            pltpu.SemaphoreType.DMA,
            pltpu.SemaphoreType.DMA,
            pltpu.SemaphoreType.DMA,
            pltpu.SemaphoreType.DMA,
        ],
    )(h0_pk, h1_pk, src_idx, dst_idx)

    out = pl.pallas_call(
        _sigmoid_body,
        out_shape=jax.ShapeDtypeStruct((E // D, D), jnp.float32),
    )(p0.reshape(E // D, D), p1.reshape(E // D, D))
    return out.reshape(E)


def kernel(z_user, z_item, edge_index):
    zu_pk = lax.bitcast_convert_type(
        z_user.astype(jnp.bfloat16).reshape(N, PW, 2), jnp.int32)
    zi_pk = lax.bitcast_convert_type(
        z_item.astype(jnp.bfloat16).reshape(N, PW, 2), jnp.int32)
    zf_pk = jnp.stack([zu_pk, zi_pk], axis=1).reshape(N2, PW)
    h0_pk = zf_pk[:, :HPW]
    h1_pk = zf_pk[:, HPW:]
    src_idx = edge_index[0].astype(jnp.int32) * 2
    dst_idx = edge_index[1].astype(jnp.int32) * 2 + 1
    return _edge_decoder(h0_pk, h1_pk, src_idx, dst_idx)
